# Initial kernel scaffold; baseline (speedup 1.0000x reference)
import jax
import jax.numpy as jnp
from jax.experimental import pallas as pl


def kernel(x):
    raise NotImplementedError("write your pallas kernel here")


# speedup vs baseline: 2.0862x; 2.0862x over previous
import functools

import jax
import jax.numpy as jnp
from jax import lax
from jax.experimental import pallas as pl
from jax.experimental.pallas import tpu as pltpu
from jax.experimental.pallas import tpu_sc as plsc

G = 16
NVOX = G * G * G
NFEAT = 10
NPTS = 2_000_000
NC, NS, L = 2, 16, 16
NW = NC * NS
BLK_PTS = 4000
BLK_F = BLK_PTS * 3
NBLK = NPTS // BLK_PTS
ROUNDS = -(-NBLK // NW)
GROUPS = BLK_PTS // L
CHUNK = NVOX // NW

_mesh = functools.partial(
    plsc.VectorSubcoreMesh, core_axis_name="c", subcore_axis_name="s")


def _wid():
    return lax.axis_index("s") * NC + lax.axis_index("c")


def _lanes():
    return lax.iota(jnp.int32, L)


_GDN = lax.GatherDimensionNumbers(
    offset_dims=(), collapsed_slice_dims=(0,), start_index_map=(0,))


def _perm(v, idx):
    return lax.gather(v, idx[:, None], dimension_numbers=_GDN,
                      slice_sizes=(1,),
                      mode=lax.GatherScatterMode.PROMISE_IN_BOUNDS)


def _bmax(v):
    ln = _lanes()
    for s in (1, 2, 4, 8):
        v = jnp.maximum(v, _perm(v, (ln + s) % L))
    return v


def _bmin(v):
    return -_bmax(-v)


def _minmax_body(x_hbm, out_hbm, buf, stage, sem):
    wid = _wid()
    ln = _lanes()
    inf = jnp.full((L,), jnp.inf, jnp.float32)
    accs = [inf, inf, inf, inf, inf, inf]

    for i in range(ROUNDS):
        blk = wid + i * NW

        @pl.when(blk < NBLK)
        def _():
            off = pl.multiple_of(blk * BLK_F, 8)
            pltpu.sync_copy(x_hbm.at[pl.ds(off, BLK_F)], buf)

        def g_body(g, carry):
            a0, a1, a2, b0, b1, b2 = carry
            base = g * (3 * L)
            v0 = buf[pl.ds(base, L)]
            v1 = buf[pl.ds(base + L, L)]
            v2 = buf[pl.ds(base + 2 * L, L)]
            return (jnp.minimum(a0, v0), jnp.minimum(a1, v1),
                    jnp.minimum(a2, v2), jnp.minimum(b0, -v0),
                    jnp.minimum(b1, -v1), jnp.minimum(b2, -v2))

        n_g = jnp.where(blk < NBLK, GROUPS, 0)
        accs[:] = list(lax.fori_loop(0, n_g, g_body, tuple(accs)))

    row = inf
    for c in range(3):
        mn = inf
        ng = inf
        for p in range(3):
            m = ((L * p + ln) % 3) == c
            mn = jnp.minimum(mn, jnp.where(m, accs[p], inf))
            ng = jnp.minimum(ng, jnp.where(m, accs[3 + p], inf))
        row = jnp.where(ln == c, _bmin(mn), row)
        row = jnp.where(ln == c + 3, _bmin(ng), row)
    stage[...] = row
    pltpu.sync_copy(stage, out_hbm.at[wid])


def _minmax(xf):
    return pl.kernel(
        _minmax_body,
        out_type=jax.ShapeDtypeStruct((NW, L), jnp.float32),
        mesh=_mesh(),
        compiler_params=pltpu.CompilerParams(needs_layout_passes=False),
        scratch_types=[
            pltpu.VMEM((BLK_F,), jnp.float32),
            pltpu.VMEM((L,), jnp.float32),
            pltpu.SemaphoreType.DMA,
        ],
    )(xf)


def _bin_body(x_hbm, mm_hbm, out_hbm, buf, acc, mmv, sem):
    wid = _wid()
    ln = _lanes()
    inf = jnp.full((L,), jnp.inf, jnp.float32)

    pltpu.sync_copy(mm_hbm, mmv)
    macc = inf
    for r in range(NW):
        macc = jnp.minimum(macc, mmv[r])
    minv = [_bmin(jnp.where(ln == c, macc, inf)) for c in range(3)]
    maxv = [-_bmin(jnp.where(ln == c + 3, macc, inf)) for c in range(3)]
    extv = [jnp.maximum(maxv[c] - minv[c], 1e-6) for c in range(3)]

    zero = jnp.zeros((L,), jnp.float32)

    def z_body(j, _):
        acc[pl.ds(j * L, L)] = zero
        return 0

    lax.fori_loop(0, NVOX * NFEAT // L, z_body, 0)

    ix = ln * 3
    iy = ix + 1
    iz = ix + 2
    ones = jnp.ones((L,), jnp.float32)
    gf = jnp.float32(float(G))

    for i in range(ROUNDS):
        blk = wid + i * NW

        @pl.when(blk < NBLK)
        def _():
            off = pl.multiple_of(blk * BLK_F, 8)
            pltpu.sync_copy(x_hbm.at[pl.ds(off, BLK_F)], buf)

            def g_body(g, _):
                base = g * (3 * L)
                xs = plsc.load_gather(buf, [base + ix])
                ys = plsc.load_gather(buf, [base + iy])
                zs = plsc.load_gather(buf, [base + iz])
                vx = jnp.clip(((xs - minv[0]) / extv[0] * gf)
                              .astype(jnp.int32), 0, G - 1)
                vy = jnp.clip(((ys - minv[1]) / extv[1] * gf)
                              .astype(jnp.int32), 0, G - 1)
                vz = jnp.clip(((zs - minv[2]) / extv[2] * gf)
                              .astype(jnp.int32), 0, G - 1)
                addr = ((vx * G + vy) * G + vz) * NFEAT
                feats = (ones, xs, ys, zs, xs * xs, xs * ys, xs * zs,
                         ys * ys, ys * zs, zs * zs)
                for j, f in enumerate(feats):
                    plsc.addupdate_scatter(acc, [addr + j], f)
                return 0

            lax.fori_loop(0, GROUPS, g_body, 0)

    pltpu.sync_copy(acc, out_hbm.at[wid])


def _bin(xf, mm):
    return pl.kernel(
        _bin_body,
        out_type=jax.ShapeDtypeStruct((NW, NVOX * NFEAT), jnp.float32),
        mesh=_mesh(),
        compiler_params=pltpu.CompilerParams(needs_layout_passes=False),
        scratch_types=[
            pltpu.VMEM((BLK_F,), jnp.float32),
            pltpu.VMEM((NVOX * NFEAT,), jnp.float32),
            pltpu.VMEM((NW, L), jnp.float32),
            pltpu.SemaphoreType.DMA,
        ],
    )(xf, mm)


def _final_body(part_hbm, means_hbm, covs_hbm, pbuf, accv, mst, cst, sem):
    wid = _wid()
    ln = _lanes()
    span = CHUNK * NFEAT
    off = pl.multiple_of(wid * span, 8)

    pltpu.sync_copy(part_hbm.at[0, pl.ds(off, span)], accv)
    for t in range(1, NW):
        pltpu.sync_copy(part_hbm.at[t, pl.ds(off, span)], pbuf)

        def a_body(j, _):
            sl = pl.ds(j * L, L)
            accv[sl] = accv[sl] + pbuf[sl]
            return 0

        lax.fori_loop(0, span // L, a_body, 0)

    for g in range(CHUNK // L):
        vbase = g * (L * NFEAT) + ln * NFEAT
        f = [plsc.load_gather(accv, [vbase + j]) for j in range(NFEAT)]
        cnt = jnp.maximum(f[0], 1.0)
        mx, my, mz = f[1] / cnt, f[2] / cnt, f[3] / cnt
        cxx = f[4] / cnt - mx * mx
        cxy = f[5] / cnt - mx * my
        cxz = f[6] / cnt - mx * mz
        cyy = f[7] / cnt - my * my
        cyz = f[8] / cnt - my * mz
        czz = f[9] / cnt - mz * mz
        midx = g * (L * 3) + ln * 3
        for c, v in enumerate((mx, my, mz)):
            plsc.store_scatter(mst, [midx + c], v)
        cidx = g * (L * 9) + ln * 9
        cov = (cxx, cxy, cxz, cxy, cyy, cyz, cxz, cyz, czz)
        for k, v in enumerate(cov):
            plsc.store_scatter(cst, [cidx + k], v)

    moff = pl.multiple_of(wid * CHUNK * 3, 8)
    coff = pl.multiple_of(wid * CHUNK * 9, 8)
    pltpu.sync_copy(mst, means_hbm.at[pl.ds(moff, CHUNK * 3)])
    pltpu.sync_copy(cst, covs_hbm.at[pl.ds(coff, CHUNK * 9)])


def _finalize(parts):
    return pl.kernel(
        _final_body,
        out_type=(
            jax.ShapeDtypeStruct((NVOX * 3,), jnp.float32),
            jax.ShapeDtypeStruct((NVOX * 9,), jnp.float32),
        ),
        mesh=_mesh(),
        compiler_params=pltpu.CompilerParams(needs_layout_passes=False),
        scratch_types=[
            pltpu.VMEM((CHUNK * NFEAT,), jnp.float32),
            pltpu.VMEM((CHUNK * NFEAT,), jnp.float32),
            pltpu.VMEM((CHUNK * 3,), jnp.float32),
            pltpu.VMEM((CHUNK * 9,), jnp.float32),
            pltpu.SemaphoreType.DMA,
        ],
    )(parts)


def kernel(x):
    xf = x.reshape(-1)
    mm = _minmax(xf)
    parts = _bin(xf, mm)
    means, covs = _finalize(parts)
    return means, covs


# speedup vs baseline: 35.1408x; 16.8445x over previous
import functools

import jax
import jax.numpy as jnp
from jax import lax
from jax.experimental import pallas as pl
from jax.experimental.pallas import tpu as pltpu
from jax.experimental.pallas import tpu_sc as plsc

G = 16
NVOX = G * G * G
NFEAT = 10
NPTS = 2_000_000
NC, NS, L = 2, 16, 16
NW = NC * NS
BLK = 4000
NBLK = NPTS // BLK
ROUNDS = -(-NBLK // NW)
GROUPS = BLK // L
CHUNK = NVOX // NW

_mesh = functools.partial(
    plsc.VectorSubcoreMesh, core_axis_name="c", subcore_axis_name="s")
_params = pltpu.CompilerParams(needs_layout_passes=False)


def _wid():
    return lax.axis_index("s") * NC + lax.axis_index("c")


def _lanes():
    return lax.iota(jnp.int32, L)


_GDN = lax.GatherDimensionNumbers(
    offset_dims=(), collapsed_slice_dims=(0,), start_index_map=(0,))


def _perm(v, idx):
    return lax.gather(v, idx[:, None], dimension_numbers=_GDN,
                      slice_sizes=(1,),
                      mode=lax.GatherScatterMode.PROMISE_IN_BOUNDS)


def _bmax(v):
    ln = _lanes()
    for s in (1, 2, 4, 8):
        v = jnp.maximum(v, _perm(v, (ln + s) % L))
    return v


def _bmin(v):
    return -_bmax(-v)


def _minmax_body(x0, x1, x2, out_hbm, b0, b1, b2, stage, sem):
    wid = _wid()
    ln = _lanes()
    inf = jnp.full((L,), jnp.inf, jnp.float32)
    accs = [inf] * 6

    for i in range(ROUNDS):
        blk = wid + i * NW

        @pl.when(blk < NBLK)
        def _():
            off = pl.multiple_of(blk * BLK, 8)
            pltpu.sync_copy(x0.at[pl.ds(off, BLK)], b0)
            pltpu.sync_copy(x1.at[pl.ds(off, BLK)], b1)
            pltpu.sync_copy(x2.at[pl.ds(off, BLK)], b2)

        def g_body(g, carry):
            a0, a1, a2, n0, n1, n2 = carry
            sl = pl.ds(g * L, L)
            v0, v1, v2 = b0[sl], b1[sl], b2[sl]
            return (jnp.minimum(a0, v0), jnp.minimum(a1, v1),
                    jnp.minimum(a2, v2), jnp.minimum(n0, -v0),
                    jnp.minimum(n1, -v1), jnp.minimum(n2, -v2))

        n_g = jnp.where(blk < NBLK, GROUPS, 0)
        accs[:] = list(lax.fori_loop(0, n_g, g_body, tuple(accs)))

    row = inf
    for c in range(3):
        row = jnp.where(ln == c, _bmin(accs[c]), row)
        row = jnp.where(ln == c + 3, _bmin(accs[3 + c]), row)
    stage[...] = row
    pltpu.sync_copy(stage, out_hbm.at[wid])


def _minmax(x0, x1, x2):
    return pl.kernel(
        _minmax_body,
        out_type=jax.ShapeDtypeStruct((NW, L), jnp.float32),
        mesh=_mesh(),
        compiler_params=_params,
        scratch_types=[
            pltpu.VMEM((BLK,), jnp.float32),
            pltpu.VMEM((BLK,), jnp.float32),
            pltpu.VMEM((BLK,), jnp.float32),
            pltpu.VMEM((L,), jnp.float32),
            pltpu.SemaphoreType.DMA,
        ],
    )(x0, x1, x2)


def _bin_body(x0, x1, x2, mm_hbm, out_hbm, b0, b1, b2, acc, mmv, sem):
    wid = _wid()
    ln = _lanes()
    inf = jnp.full((L,), jnp.inf, jnp.float32)

    pltpu.sync_copy(mm_hbm, mmv)
    macc = inf
    for r in range(NW):
        macc = jnp.minimum(macc, mmv[r])
    minv = [_bmin(jnp.where(ln == c, macc, inf)) for c in range(3)]
    maxv = [-_bmin(jnp.where(ln == c + 3, macc, inf)) for c in range(3)]
    extv = [jnp.maximum(maxv[c] - minv[c], 1e-6) for c in range(3)]

    zero = jnp.zeros((L,), jnp.float32)

    def z_body(j, _):
        acc[pl.ds(j * L, L)] = zero
        return 0

    lax.fori_loop(0, NVOX * NFEAT // L, z_body, 0)

    ones = jnp.ones((L,), jnp.float32)
    gf = jnp.float32(float(G))

    for i in range(ROUNDS):
        blk = wid + i * NW

        @pl.when(blk < NBLK)
        def _():
            off = pl.multiple_of(blk * BLK, 8)
            pltpu.sync_copy(x0.at[pl.ds(off, BLK)], b0)
            pltpu.sync_copy(x1.at[pl.ds(off, BLK)], b1)
            pltpu.sync_copy(x2.at[pl.ds(off, BLK)], b2)

            def g_body(g, _):
                sl = pl.ds(g * L, L)
                xs, ys, zs = b0[sl], b1[sl], b2[sl]
                vx = jnp.clip(((xs - minv[0]) / extv[0] * gf)
                              .astype(jnp.int32), 0, G - 1)
                vy = jnp.clip(((ys - minv[1]) / extv[1] * gf)
                              .astype(jnp.int32), 0, G - 1)
                vz = jnp.clip(((zs - minv[2]) / extv[2] * gf)
                              .astype(jnp.int32), 0, G - 1)
                addr = ((vx * G + vy) * G + vz) * NFEAT
                feats = (ones, xs, ys, zs, xs * xs, xs * ys, xs * zs,
                         ys * ys, ys * zs, zs * zs)
                for j, f in enumerate(feats):
                    plsc.addupdate_scatter(acc, [addr + j], f)
                return 0

            lax.fori_loop(0, GROUPS, g_body, 0)

    pltpu.sync_copy(acc, out_hbm.at[wid])


def _bin(x0, x1, x2, mm):
    return pl.kernel(
        _bin_body,
        out_type=jax.ShapeDtypeStruct((NW, NVOX * NFEAT), jnp.float32),
        mesh=_mesh(),
        compiler_params=_params,
        scratch_types=[
            pltpu.VMEM((BLK,), jnp.float32),
            pltpu.VMEM((BLK,), jnp.float32),
            pltpu.VMEM((BLK,), jnp.float32),
            pltpu.VMEM((NVOX * NFEAT,), jnp.float32),
            pltpu.VMEM((NW, L), jnp.float32),
            pltpu.SemaphoreType.DMA,
        ],
    )(x0, x1, x2, mm)


def _final_body(part_hbm, means_hbm, covs_hbm, pbuf, accv, mst, cst, sem):
    wid = _wid()
    ln = _lanes()
    span = CHUNK * NFEAT
    off = pl.multiple_of(wid * span, 8)

    pltpu.sync_copy(part_hbm.at[0, pl.ds(off, span)], accv)
    for t in range(1, NW):
        pltpu.sync_copy(part_hbm.at[t, pl.ds(off, span)], pbuf)

        def a_body(j, _):
            sl = pl.ds(j * L, L)
            accv[sl] = accv[sl] + pbuf[sl]
            return 0

        lax.fori_loop(0, span // L, a_body, 0)

    for g in range(CHUNK // L):
        vbase = g * (L * NFEAT) + ln * NFEAT
        f = [plsc.load_gather(accv, [vbase + j]) for j in range(NFEAT)]
        cnt = jnp.maximum(f[0], 1.0)
        mx, my, mz = f[1] / cnt, f[2] / cnt, f[3] / cnt
        cxx = f[4] / cnt - mx * mx
        cxy = f[5] / cnt - mx * my
        cxz = f[6] / cnt - mx * mz
        cyy = f[7] / cnt - my * my
        cyz = f[8] / cnt - my * mz
        czz = f[9] / cnt - mz * mz
        midx = g * (L * 3) + ln * 3
        for c, v in enumerate((mx, my, mz)):
            plsc.store_scatter(mst, [midx + c], v)
        cidx = g * (L * 9) + ln * 9
        cov = (cxx, cxy, cxz, cxy, cyy, cyz, cxz, cyz, czz)
        for k, v in enumerate(cov):
            plsc.store_scatter(cst, [cidx + k], v)

    moff = pl.multiple_of(wid * CHUNK * 3, 8)
    coff = pl.multiple_of(wid * CHUNK * 9, 8)
    pltpu.sync_copy(mst, means_hbm.at[pl.ds(moff, CHUNK * 3)])
    pltpu.sync_copy(cst, covs_hbm.at[pl.ds(coff, CHUNK * 9)])


def _finalize(parts):
    return pl.kernel(
        _final_body,
        out_type=(
            jax.ShapeDtypeStruct((NVOX * 3,), jnp.float32),
            jax.ShapeDtypeStruct((NVOX * 9,), jnp.float32),
        ),
        mesh=_mesh(),
        compiler_params=_params,
        scratch_types=[
            pltpu.VMEM((CHUNK * NFEAT,), jnp.float32),
            pltpu.VMEM((CHUNK * NFEAT,), jnp.float32),
            pltpu.VMEM((CHUNK * 3,), jnp.float32),
            pltpu.VMEM((CHUNK * 9,), jnp.float32),
            pltpu.SemaphoreType.DMA,
        ],
    )(parts)


def kernel(x):
    x0 = x[:, 0]
    x1 = x[:, 1]
    x2 = x[:, 2]
    mm = _minmax(x0, x1, x2)
    parts = _bin(x0, x1, x2, mm)
    means, covs = _finalize(parts)
    return means, covs


# speedup vs baseline: 102.5607x; 2.9186x over previous
import functools

import jax
import jax.numpy as jnp
from jax import lax
from jax.experimental import pallas as pl
from jax.experimental.pallas import tpu as pltpu
from jax.experimental.pallas import tpu_sc as plsc

G = 16
NVOX = G * G * G
NFEAT = 10
STRIDE = 11
NPTS = 2_000_000
NC, NS, L = 2, 16, 16
NW = NC * NS

ROWS = NPTS // 128
RPAD = 16000
WR = 800
TCGRID = RPAD // WR
TCW = WR * 128

BROWS = 32
BLK = BROWS * 128
NBLK = -(-ROWS // BROWS)
ROUNDS = -(-NBLK // NW)
GROUPS = BLK // L
CHUNK = NVOX // NW

_mesh = functools.partial(
    plsc.VectorSubcoreMesh, core_axis_name="c", subcore_axis_name="s")
_params = pltpu.CompilerParams(needs_layout_passes=False)


def _wid():
    return lax.axis_index("s") * NC + lax.axis_index("c")


def _lanes():
    return lax.iota(jnp.int32, L)


_GDN = lax.GatherDimensionNumbers(
    offset_dims=(), collapsed_slice_dims=(0,), start_index_map=(0,))


def _perm(v, idx):
    return lax.gather(v, idx[:, None], dimension_numbers=_GDN,
                      slice_sizes=(1,),
                      mode=lax.GatherScatterMode.PROMISE_IN_BOUNDS)


def _bmax(v):
    ln = _lanes()
    for s in (1, 2, 4, 8):
        v = jnp.maximum(v, _perm(v, (ln + s) % L))
    return v


def _bmin(v):
    return -_bmax(-v)


def _tc_body(xt_ref, x0_ref, x1_ref, x2_ref, mm_ref, smin_ref):
    i = pl.program_id(0)
    b = xt_ref[...]
    r = b.reshape(3, WR, 128)
    x0_ref[...] = r[0]
    x1_ref[...] = r[1]
    x2_ref[...] = r[2]
    ridx = lax.broadcasted_iota(jnp.int32, (1, WR, 128), 1)
    lidx = lax.broadcasted_iota(jnp.int32, (1, WR, 128), 2)
    valid = i * TCW + ridx * 128 + lidx < NPTS
    inf3 = jnp.full((3, WR, 128), jnp.inf, jnp.float32)
    bmin = jnp.min(jnp.where(valid, r, inf3), axis=1)
    bneg = jnp.min(jnp.where(valid, -r, inf3), axis=1)
    upd = jnp.concatenate(
        [bmin, bneg, jnp.full((2, 128), jnp.inf, jnp.float32)], axis=0)

    @pl.when(i == 0)
    def _():
        smin_ref[...] = jnp.full((8, 128), jnp.inf, jnp.float32)

    smin_ref[...] = jnp.minimum(smin_ref[...], upd)

    @pl.when(i == TCGRID - 1)
    def _():
        mm_ref[...] = smin_ref[...]


def _tc_split(xt):
    return pl.pallas_call(
        _tc_body,
        grid=(TCGRID,),
        in_specs=[pl.BlockSpec((3, TCW), lambda i: (0, i))],
        out_specs=[
            pl.BlockSpec((WR, 128), lambda i: (i, 0)),
            pl.BlockSpec((WR, 128), lambda i: (i, 0)),
            pl.BlockSpec((WR, 128), lambda i: (i, 0)),
            pl.BlockSpec((8, 128), lambda i: (0, 0)),
        ],
        out_shape=[
            jax.ShapeDtypeStruct((RPAD, 128), jnp.float32),
            jax.ShapeDtypeStruct((RPAD, 128), jnp.float32),
            jax.ShapeDtypeStruct((RPAD, 128), jnp.float32),
            jax.ShapeDtypeStruct((8, 128), jnp.float32),
        ],
        scratch_shapes=[pltpu.VMEM((8, 128), jnp.float32)],
    )(xt)


def _bin_body(x0, x1, x2, mm_hbm, out_hbm,
              b0a, b1a, b2a, b0b, b1b, b2b, acc, mmv, sem0, sem1):
    wid = _wid()
    ln = _lanes()
    inf = jnp.full((L,), jnp.inf, jnp.float32)
    planes = (x0, x1, x2)
    bufs = ((b0a, b1a, b2a), (b0b, b1b, b2b))
    sems = (sem0, sem1)

    pltpu.sync_copy(mm_hbm, mmv)

    def _rowmin(row):
        m = inf
        for k in range(128 // L):
            m = jnp.minimum(m, mmv[row, pl.ds(k * L, L)])
        return _bmin(m)

    minv = [_rowmin(c) for c in range(3)]
    maxv = [-_rowmin(c + 3) for c in range(3)]
    extv = [jnp.maximum(maxv[c] - minv[c], 1e-6) for c in range(3)]

    zero = jnp.zeros((L,), jnp.float32)

    @plsc.parallel_loop(0, NVOX * STRIDE // L, unroll=8)
    def _(j):
        acc[pl.ds(j * L, L)] = zero

    ones = jnp.ones((L,), jnp.float32)
    gf = jnp.float32(float(G))

    def _copies(i):
        blk = wid + i * NW
        row = blk * BROWS
        par = i % 2
        return blk, [pltpu.make_async_copy(
            planes[p].at[pl.ds(row, BROWS), :], bufs[par][p], sems[par])
            for p in range(3)]

    def issue(i):
        blk, cps = _copies(i)

        @pl.when(blk < NBLK)
        def _():
            for cp in cps:
                cp.start()

    issue(0)
    for i in range(ROUNDS):
        if i + 1 < ROUNDS:
            issue(i + 1)
        blk, cps = _copies(i)
        b0, b1, b2 = bufs[i % 2]

        @pl.when(blk < NBLK)
        def _():
            for cp in cps:
                cp.wait()

            @plsc.parallel_loop(0, GROUPS, unroll=8)
            def _(g):
                r = lax.shift_right_logical(g, 3)
                sl = pl.ds((g & 7) * L, L)
                m = ln < jnp.where(blk * BROWS + r < ROWS, L, 0)
                xs, ys, zs = b0[r, sl], b1[r, sl], b2[r, sl]
                vx = jnp.clip(((xs - minv[0]) / extv[0] * gf)
                              .astype(jnp.int32), 0, G - 1)
                vy = jnp.clip(((ys - minv[1]) / extv[1] * gf)
                              .astype(jnp.int32), 0, G - 1)
                vz = jnp.clip(((zs - minv[2]) / extv[2] * gf)
                              .astype(jnp.int32), 0, G - 1)
                addr = ((vx * G + vy) * G + vz) * STRIDE
                feats = (ones, xs, ys, zs, xs * xs, xs * ys, xs * zs,
                         ys * ys, ys * zs, zs * zs)
                for j, f in enumerate(feats):
                    plsc.addupdate_scatter(acc, [addr + j], f, mask=m)

    pltpu.sync_copy(acc, out_hbm.at[wid])


def _bin(x0, x1, x2, mm):
    return pl.kernel(
        _bin_body,
        out_type=jax.ShapeDtypeStruct((NW, NVOX * STRIDE), jnp.float32),
        mesh=_mesh(),
        compiler_params=_params,
        scratch_types=[
            pltpu.VMEM((BROWS, 128), jnp.float32),
            pltpu.VMEM((BROWS, 128), jnp.float32),
            pltpu.VMEM((BROWS, 128), jnp.float32),
            pltpu.VMEM((BROWS, 128), jnp.float32),
            pltpu.VMEM((BROWS, 128), jnp.float32),
            pltpu.VMEM((BROWS, 128), jnp.float32),
            pltpu.VMEM((NVOX * STRIDE,), jnp.float32),
            pltpu.VMEM((8, 128), jnp.float32),
            pltpu.SemaphoreType.DMA,
            pltpu.SemaphoreType.DMA,
        ],
    )(x0, x1, x2, mm)


def _final_body(part_hbm, means_hbm, covs_hbm, pbuf, accv, mst, cst, sem):
    wid = _wid()
    ln = _lanes()
    span = CHUNK * STRIDE
    off = pl.multiple_of(wid * span, 8)

    cps = [pltpu.make_async_copy(part_hbm.at[t, pl.ds(off, span)],
                                 pbuf.at[pl.ds((t - 1) * span, span)], sem)
           for t in range(1, NW)]
    for cp in cps:
        cp.start()
    pltpu.sync_copy(part_hbm.at[0, pl.ds(off, span)], accv)
    for cp in cps:
        cp.wait()

    @plsc.parallel_loop(0, span // L, unroll=2)
    def _(j):
        sl = pl.ds(j * L, L)
        a = accv[sl]
        for t in range(1, NW):
            a = a + pbuf[pl.ds((t - 1) * span + j * L, L)]
        accv[sl] = a

    for g in range(CHUNK // L):
        vbase = g * (L * STRIDE) + ln * STRIDE
        f = [plsc.load_gather(accv, [vbase + j]) for j in range(NFEAT)]
        cnt = jnp.maximum(f[0], 1.0)
        mx, my, mz = f[1] / cnt, f[2] / cnt, f[3] / cnt
        cxx = f[4] / cnt - mx * mx
        cxy = f[5] / cnt - mx * my
        cxz = f[6] / cnt - mx * mz
        cyy = f[7] / cnt - my * my
        cyz = f[8] / cnt - my * mz
        czz = f[9] / cnt - mz * mz
        midx = g * (L * 3) + ln * 3
        for c, v in enumerate((mx, my, mz)):
            plsc.store_scatter(mst, [midx + c], v)
        cidx = g * (L * 9) + ln * 9
        cov = (cxx, cxy, cxz, cxy, cyy, cyz, cxz, cyz, czz)
        for k, v in enumerate(cov):
            plsc.store_scatter(cst, [cidx + k], v)

    moff = pl.multiple_of(wid * CHUNK * 3, 8)
    coff = pl.multiple_of(wid * CHUNK * 9, 8)
    pltpu.sync_copy(mst, means_hbm.at[pl.ds(moff, CHUNK * 3)])
    pltpu.sync_copy(cst, covs_hbm.at[pl.ds(coff, CHUNK * 9)])


def _finalize(parts):
    return pl.kernel(
        _final_body,
        out_type=(
            jax.ShapeDtypeStruct((NVOX * 3,), jnp.float32),
            jax.ShapeDtypeStruct((NVOX * 9,), jnp.float32),
        ),
        mesh=_mesh(),
        compiler_params=_params,
        scratch_types=[
            pltpu.VMEM(((NW - 1) * CHUNK * STRIDE,), jnp.float32),
            pltpu.VMEM((CHUNK * STRIDE,), jnp.float32),
            pltpu.VMEM((CHUNK * 3,), jnp.float32),
            pltpu.VMEM((CHUNK * 9,), jnp.float32),
            pltpu.SemaphoreType.DMA,
        ],
    )(parts)


def kernel(x):
    x0, x1, x2, mm = _tc_split(x.T)
    parts = _bin(x0, x1, x2, mm)
    means, covs = _finalize(parts)
    return means, covs
